# baseline (speedup 1.0000x reference)
import functools

import jax
import jax.numpy as jnp
import numpy as np
from jax import lax
from jax.experimental import pallas as pl
from jax.experimental.pallas import tpu as pltpu
from jax.experimental.pallas import tpu_sc as plsc

N = 1000
E = 32000
EP = 32768
EMB = 64
OBS = 6
NOBS = 32

EBLK = 4096
NGRID = EP // EBLK

_NC, _NS, _L = 2, 16, 16
_NW = _NC * _NS
_EPW = EP // _NW
_SPW = E // _NW


def _ln(g, b, x):
    m = x.mean(-1, keepdims=True)
    v = ((x - m) ** 2).mean(-1, keepdims=True)
    return (x - m) / jnp.sqrt(v + 1e-5) * g + b


def _attn_block(p, x, kk, vv):
    q = x @ p['q']['w'] + p['q']['b']
    s = q @ kk.T / np.sqrt(EMB)
    s = s - s.max(-1, keepdims=True)
    es = jnp.exp(s)
    attn = es / es.sum(-1, keepdims=True)
    x = _ln(p['ln1']['g'], p['ln1']['b'], x + attn @ vv)
    h = jnp.maximum(x @ p['mlp']['l1']['w'] + p['mlp']['l1']['b'], 0.0)
    h = h @ p['mlp']['l2']['w'] + p['mlp']['l2']['b']
    return _ln(p['ln2']['g'], p['ln2']['b'], x + h)


def _pad_mask(v, fill):
    i = pl.program_id(0)
    rows = lax.broadcasted_iota(jnp.int32, v.shape, 0) + i * EBLK
    return jnp.where(rows < E, v, fill)


def _node_kernel(v_ref, lab_ref, obs_ref, pe_ref, src_ref, dst_ref, w_ref,
                 x_ref, stab_ref, dtab_ref, oec_ref, keys_ref):
    w = jax.tree.map(lambda r: r[...], w_ref)
    vc = jnp.concatenate([v_ref[...], lab_ref[...]], axis=-1)
    lab0 = lab_ref[...][:, 0]
    gw = (lab0 == 1.0).astype(jnp.float32)
    goal = gw[None, :] @ vc
    d = vc - goal
    feat = jnp.concatenate([vc, jnp.broadcast_to(goal, vc.shape), d, d * d], -1)
    h = jnp.maximum(feat @ w['hx']['l1']['w'] + w['hx']['l1']['b'], 0.0)
    x = h @ w['hx']['l2']['w'] + w['hx']['l2']['b']

    obs = obs_ref[...]
    pe = pe_ref[...]
    h = jnp.maximum(obs @ w['onc']['l1']['w'] + w['onc']['l1']['b'], 0.0)
    onc = h @ w['onc']['l2']['w'] + w['onc']['l2']['b'] + pe
    h = jnp.maximum(obs @ w['oec']['l1']['w'] + w['oec']['l1']['b'], 0.0)
    oec = h @ w['oec']['l2']['w'] + w['oec']['l2']['b'] + pe
    oec_ref[...] = oec

    for p in w['node_attn']:
        kk = onc @ p['k']['w'] + p['k']['b']
        vv = onc @ p['v']['w'] + p['v']['b']
        x = _attn_block(p, x, kk, vv)
    x_ref[...] = x

    q_t = vc @ w['hy_wca']
    p_t = vc @ w['hy_wab']
    a0 = x @ w['fx_wab']
    b0 = x @ w['fx_wca']
    stab_ref[...] = jnp.concatenate([q_t, a0], axis=-1)
    dtab_ref[...] = jnp.concatenate([p_t, b0], axis=-1)
    keys_ref[...] = src_ref[...] * N + dst_ref[...]


def _edge_kernel(gs_ref, gd_ref, w_ref, y_ref, msg_ref):
    w = jax.tree.map(lambda r: r[...], w_ref)
    gs = gs_ref[...]
    gd = gd_ref[...]
    h = jnp.maximum(gd[:, :EMB] + gs[:, :EMB] + w['hy_b1'], 0.0)
    y = h @ w['hy']['l2']['w'] + w['hy']['l2']['b']
    oec = w['oec']
    for p in w['edge_attn']:
        kk = oec @ p['k']['w'] + p['k']['b']
        vv = oec @ p['v']['w'] + p['v']['b']
        y = _attn_block(p, y, kk, vv)
    y_ref[...] = y
    u = jnp.maximum(gs[:, EMB:] + gd[:, EMB:] + y @ w['fx_wd'] + w['fx_b1'], 0.0)
    msg_ref[...] = _pad_mask(u @ w['fx']['l2']['w'] + w['fx']['l2']['b'],
                             -jnp.inf)


def _nodeupd_kernel(accs_ref, x_ref, w_ref, xo_ref, stab_ref, dtab_ref):
    w = jax.tree.map(lambda r: r[...], w_ref)
    agg2 = jnp.max(accs_ref[...], axis=0)
    agg = jnp.concatenate([agg2[:, :EMB], agg2[:, EMB:]], axis=0)
    agg = jnp.where(jnp.isneginf(agg), 0.0, agg)
    x = jnp.maximum(x_ref[...], agg)
    xo_ref[...] = x
    stab_ref[...] = jnp.concatenate([x @ w['fx_wab'], x @ w['fy_wca']], -1)
    dtab_ref[...] = jnp.concatenate([x @ w['fx_wca'], x @ w['fy_wab']], -1)


def _ym_kernel(y_ref, gs_ref, gd_ref, w_ref, yo_ref, msg_ref):
    w = jax.tree.map(lambda r: r[...], w_ref)
    gs = gs_ref[...]
    gd = gd_ref[...]
    t = jnp.maximum(gd[:, EMB:] + gs[:, EMB:] + w['fy_b1'], 0.0)
    y = jnp.maximum(y_ref[...], t @ w['fy']['l2']['w'] + w['fy']['l2']['b'])
    yo_ref[...] = y
    u = jnp.maximum(gs[:, :EMB] + gd[:, :EMB] + y @ w['fx_wd'] + w['fx_b1'], 0.0)
    msg_ref[...] = _pad_mask(u @ w['fx']['l2']['w'] + w['fx']['l2']['b'],
                             -jnp.inf)


def _yfin_kernel(y_ref, gs_ref, gd_ref, w_ref, yo_ref):
    w = jax.tree.map(lambda r: r[...], w_ref)
    t = jnp.maximum(gd_ref[...][:, EMB:] + gs_ref[...][:, EMB:] + w['fy_b1'],
                    0.0)
    yn = jnp.maximum(y_ref[...], t @ w['fy']['l2']['w'] + w['fy']['l2']['b'])
    yo_ref[...] = jnp.concatenate(
        [yn, jnp.zeros((EBLK, EMB), jnp.float32)], axis=-1)


def _wspec(tree):
    return jax.tree.map(
        lambda x: pl.BlockSpec(x.shape, lambda *_: (0,) * x.ndim), tree)


def _espec(width):
    return pl.BlockSpec((EBLK, width), lambda i: (i, 0))


def _sc_mesh():
    return plsc.VectorSubcoreMesh(core_axis_name="c", subcore_axis_name="s",
                                  num_cores=_NC, num_subcores=_NS)


@functools.lru_cache(None)
def _make_gather2():
    chunk = 512
    nchunk = _EPW // chunk

    @functools.partial(
        pl.kernel, mesh=_sc_mesh(),
        out_type=(jax.ShapeDtypeStruct((EP, 128), jnp.float32),
                  jax.ShapeDtypeStruct((EP, 128), jnp.float32)),
        scratch_types=[
            pltpu.VMEM((chunk,), jnp.int32),
            pltpu.VMEM((chunk, 128), jnp.float32),
            pltpu.SemaphoreType.DMA,
        ],
    )
    def k(stab_hbm, dtab_hbm, src_hbm, dst_hbm, gs_hbm, gd_hbm,
          idx_v, rows_v, sem):
        wid = lax.axis_index("c") * _NS + lax.axis_index("s")
        base = wid * _EPW
        for c in range(nchunk):
            off = base + c * chunk
            pltpu.sync_copy(src_hbm.at[pl.ds(off, chunk)], idx_v)
            pltpu.async_copy(stab_hbm.at[idx_v], rows_v, sem).wait()
            pltpu.sync_copy(rows_v, gs_hbm.at[pl.ds(off, chunk)])
            pltpu.sync_copy(dst_hbm.at[pl.ds(off, chunk)], idx_v)
            pltpu.async_copy(dtab_hbm.at[idx_v], rows_v, sem).wait()
            pltpu.sync_copy(rows_v, gd_hbm.at[pl.ds(off, chunk)])

    return k


@functools.lru_cache(None)
def _make_segmax():
    mchunk = 256
    nchunk = _EPW // mchunk

    @functools.partial(
        pl.kernel, mesh=_sc_mesh(),
        out_type=jax.ShapeDtypeStruct((_NW, N // 2, 2 * EMB), jnp.float32),
        scratch_types=[
            pltpu.VMEM((_EPW,), jnp.int32),
            pltpu.VMEM((mchunk, EMB), jnp.float32),
            pltpu.VMEM((N // 2, 2 * EMB), jnp.float32),
            pltpu.SemaphoreType.DMA,
        ],
    )
    def k(msg_hbm, dst_hbm, ninf_hbm, accs_hbm, idx_v, mbuf, acc, sem):
        wid = lax.axis_index("c") * _NS + lax.axis_index("s")
        base = wid * _EPW
        pltpu.sync_copy(ninf_hbm, acc)
        pltpu.sync_copy(dst_hbm.at[pl.ds(base, _EPW)], idx_v)
        for c in range(nchunk):
            pltpu.sync_copy(msg_hbm.at[pl.ds(base + c * mchunk, mchunk)], mbuf)

            def body(g, _):
                dvec = idx_v[pl.ds(c * mchunk + g * _L, _L)]
                for j in range(_L):
                    d = dvec[j]
                    hi = jnp.where(d >= N // 2, 1, 0)
                    dr = d - hi * (N // 2)
                    doff = hi * EMB
                    e = g * _L + j
                    for f in range(EMB // _L):
                        sl = pl.ds(doff + f * _L, _L)
                        msl = pl.ds(f * _L, _L)
                        acc[dr, sl] = jnp.maximum(acc[dr, sl], mbuf[e, msl])
                return 0

            lax.fori_loop(0, mchunk // _L, body, 0)
        pltpu.sync_copy(acc, accs_hbm.at[wid])

    return k


@functools.lru_cache(None)
def _make_scatter():
    @functools.partial(
        pl.kernel, mesh=_sc_mesh(),
        out_type=jax.ShapeDtypeStruct((8,), jnp.float32),
        scratch_types=[
            pltpu.VMEM((200,), jnp.int32),
            pltpu.VMEM((200, 2 * EMB), jnp.float32),
            pltpu.SemaphoreType.DMA,
        ],
    )
    def k(y_hbm, keys_hbm, base_ref, out_ref, idx_v, rows_v, sem):
        wid = lax.axis_index("c") * _NS + lax.axis_index("s")
        base = wid * _SPW
        for c in range(_SPW // 200):
            off = base + c * 200
            pltpu.sync_copy(keys_hbm.at[pl.ds(off, 200)], idx_v)
            pltpu.sync_copy(y_hbm.at[pl.ds(off, 200)], rows_v)
            pltpu.async_copy(rows_v, base_ref.at[idx_v], sem).wait()

    return k


_USE_SC_GATHER = True
_USE_SC_SEGMAX = True
_USE_SC_SCATTER = True


def _gather2(stab, dtab, src_p, dst_p):
    if _USE_SC_GATHER:
        return _make_gather2()(stab, dtab, src_p, dst_p)
    return stab[src_p], dtab[dst_p]


def kernel(v, labels, obstacles, pos_enc, edge_index, loop, params):
    del loop
    src = edge_index[0]
    dst = edge_index[1]
    pad = jnp.zeros((EP - E,), jnp.int32)
    src_p = jnp.concatenate([src, pad])
    dst_p = jnp.concatenate([dst, pad])

    hy1 = params['hy']['l1']['w']
    fx1 = params['fx']['l1']['w']
    fy1 = params['fy']['l1']['w']
    r1 = lambda b: b.reshape(1, -1)
    rall = lambda t: jax.tree.map(lambda x: r1(x) if x.ndim == 1 else x, t)

    node_w = rall({
        'hx': params['hx'], 'onc': params['obs_node_code'],
        'oec': params['obs_edge_code'], 'node_attn': params['node_attn'],
        'hy_wab': hy1[:8] + hy1[8:16], 'hy_wca': hy1[16:24] - hy1[:8],
        'fx_wab': fx1[:EMB] + fx1[EMB:2 * EMB],
        'fx_wca': fx1[2 * EMB:3 * EMB] - fx1[:EMB],
    })
    nupd_w = {
        'fx_wab': fx1[:EMB] + fx1[EMB:2 * EMB],
        'fx_wca': fx1[2 * EMB:3 * EMB] - fx1[:EMB],
        'fy_wab': fy1[:EMB] + fy1[EMB:2 * EMB],
        'fy_wca': fy1[2 * EMB:3 * EMB] - fy1[:EMB],
    }
    ym_w = rall({
        'fy': params['fy'], 'fy_b1': r1(params['fy']['l1']['b']),
        'fx': params['fx'], 'fx_b1': r1(params['fx']['l1']['b']),
        'fx_wd': fx1[3 * EMB:],
    })
    yfin_w = {'fy': ym_w['fy'], 'fy_b1': ym_w['fy_b1']}

    f32 = jnp.float32
    x0, stab, dtab, oec, keys = pl.pallas_call(
        _node_kernel,
        out_shape=(jax.ShapeDtypeStruct((N, EMB), f32),
                   jax.ShapeDtypeStruct((N, 2 * EMB), f32),
                   jax.ShapeDtypeStruct((N, 2 * EMB), f32),
                   jax.ShapeDtypeStruct((NOBS, EMB), f32),
                   jax.ShapeDtypeStruct((8, E // 8), jnp.int32)),
    )(v, labels, obstacles, pos_enc,
      src.reshape(8, E // 8), dst.reshape(8, E // 8), node_w)
    keys = keys.reshape(E)
    if _USE_SC_SCATTER:
        base = jax.new_ref(jnp.zeros((N * N, 2 * EMB), f32))

    gs, gd = _gather2(stab, dtab, src_p, dst_p)

    edge_w = rall({
        'hy': params['hy'], 'hy_b1': r1(params['hy']['l1']['b']),
        'edge_attn': params['edge_attn'], 'oec': oec,
        'fx': params['fx'], 'fx_b1': r1(params['fx']['l1']['b']),
        'fx_wd': fx1[3 * EMB:],
    })
    y, msg = pl.pallas_call(
        _edge_kernel,
        grid=(NGRID,),
        in_specs=[_espec(2 * EMB), _espec(2 * EMB), _wspec(edge_w)],
        out_specs=(_espec(EMB), _espec(EMB)),
        out_shape=(jax.ShapeDtypeStruct((EP, EMB), f32),
                   jax.ShapeDtypeStruct((EP, EMB), f32)),
    )(gs, gd, edge_w)

    ninf = jnp.full((N // 2, 2 * EMB), -jnp.inf, f32)
    nupd_call = pl.pallas_call(
        _nodeupd_kernel,
        out_shape=(jax.ShapeDtypeStruct((N, EMB), f32),
                   jax.ShapeDtypeStruct((N, 2 * EMB), f32),
                   jax.ShapeDtypeStruct((N, 2 * EMB), f32)),
    )
    ym_call = pl.pallas_call(
        _ym_kernel,
        grid=(NGRID,),
        in_specs=[_espec(EMB), _espec(2 * EMB), _espec(2 * EMB), _wspec(ym_w)],
        out_specs=(_espec(EMB), _espec(EMB)),
        out_shape=(jax.ShapeDtypeStruct((EP, EMB), f32),
                   jax.ShapeDtypeStruct((EP, EMB), f32)),
    )

    x = x0
    for it in range(3):
        if _USE_SC_SEGMAX:
            accs = _make_segmax()(msg, dst_p, ninf)
        else:
            accs = jnp.full((N, EMB), -jnp.inf, f32).at[dst_p].max(msg)
            accs = jnp.concatenate([accs[:N // 2], accs[N // 2:]], axis=1)[None]
        x, stab, dtab = nupd_call(accs, x, nupd_w)
        gs, gd = _gather2(stab, dtab, src_p, dst_p)
        if it < 2:
            y, msg = ym_call(y, gs, gd, ym_w)
        else:
            y = pl.pallas_call(
                _yfin_kernel,
                grid=(NGRID,),
                in_specs=[_espec(EMB), _espec(2 * EMB), _espec(2 * EMB),
                          _wspec(yfin_w)],
                out_specs=_espec(2 * EMB),
                out_shape=jax.ShapeDtypeStruct((EP, 2 * EMB), f32),
            )(y, gs, gd, yfin_w)

    if _USE_SC_SCATTER:
        _make_scatter()(y, keys, base)
        flat = base[...][:, :EMB]
    else:
        flat = jnp.zeros((N * N, EMB), f32).at[keys].set(y[:E, :EMB])
    return (flat.reshape(N, N, EMB), x)


# speedup vs baseline: 1.0001x; 1.0001x over previous
import functools

import jax
import jax.numpy as jnp
import numpy as np
from jax import lax
from jax.experimental import pallas as pl
from jax.experimental.pallas import tpu as pltpu
from jax.experimental.pallas import tpu_sc as plsc

N = 1000
E = 32000
EP = 32768
EMB = 64
OBS = 6
NOBS = 32

EBLK = 4096
NGRID = EP // EBLK

_NC, _NS, _L = 2, 16, 16
_NW = _NC * _NS
_EPW = EP // _NW
_SPW = E // _NW


def _ln(g, b, x):
    m = x.mean(-1, keepdims=True)
    v = ((x - m) ** 2).mean(-1, keepdims=True)
    return (x - m) / jnp.sqrt(v + 1e-5) * g + b


def _attn_block(p, x, kk, vv):
    q = x @ p['q']['w'] + p['q']['b']
    s = q @ kk.T / np.sqrt(EMB)
    s = s - s.max(-1, keepdims=True)
    es = jnp.exp(s)
    attn = es / es.sum(-1, keepdims=True)
    x = _ln(p['ln1']['g'], p['ln1']['b'], x + attn @ vv)
    h = jnp.maximum(x @ p['mlp']['l1']['w'] + p['mlp']['l1']['b'], 0.0)
    h = h @ p['mlp']['l2']['w'] + p['mlp']['l2']['b']
    return _ln(p['ln2']['g'], p['ln2']['b'], x + h)


def _pad_mask(v, fill):
    i = pl.program_id(0)
    rows = lax.broadcasted_iota(jnp.int32, v.shape, 0) + i * EBLK
    return jnp.where(rows < E, v, fill)


def _node_kernel(v_ref, lab_ref, obs_ref, pe_ref, src_ref, dst_ref, w_ref,
                 x_ref, stab_ref, dtab_ref, oec_ref, keys_ref):
    w = jax.tree.map(lambda r: r[...], w_ref)
    vc = jnp.concatenate([v_ref[...], lab_ref[...]], axis=-1)
    lab0 = lab_ref[...][:, 0]
    gw = (lab0 == 1.0).astype(jnp.float32)
    goal = gw[None, :] @ vc
    d = vc - goal
    feat = jnp.concatenate([vc, jnp.broadcast_to(goal, vc.shape), d, d * d], -1)
    h = jnp.maximum(feat @ w['hx']['l1']['w'] + w['hx']['l1']['b'], 0.0)
    x = h @ w['hx']['l2']['w'] + w['hx']['l2']['b']

    obs = obs_ref[...]
    pe = pe_ref[...]
    h = jnp.maximum(obs @ w['onc']['l1']['w'] + w['onc']['l1']['b'], 0.0)
    onc = h @ w['onc']['l2']['w'] + w['onc']['l2']['b'] + pe
    h = jnp.maximum(obs @ w['oec']['l1']['w'] + w['oec']['l1']['b'], 0.0)
    oec = h @ w['oec']['l2']['w'] + w['oec']['l2']['b'] + pe
    oec_ref[...] = oec

    for p in w['node_attn']:
        kk = onc @ p['k']['w'] + p['k']['b']
        vv = onc @ p['v']['w'] + p['v']['b']
        x = _attn_block(p, x, kk, vv)
    x_ref[...] = x

    q_t = vc @ w['hy_wca']
    p_t = vc @ w['hy_wab']
    a0 = x @ w['fx_wab']
    b0 = x @ w['fx_wca']
    stab_ref[...] = jnp.concatenate([q_t, a0], axis=-1)
    dtab_ref[...] = jnp.concatenate([p_t, b0], axis=-1)
    keys_ref[...] = src_ref[...] * N + dst_ref[...]


def _edge_kernel(gs_ref, gd_ref, w_ref, y_ref, msg_ref):
    w = jax.tree.map(lambda r: r[...], w_ref)
    gs = gs_ref[...]
    gd = gd_ref[...]
    h = jnp.maximum(gd[:, :EMB] + gs[:, :EMB] + w['hy_b1'], 0.0)
    y = h @ w['hy']['l2']['w'] + w['hy']['l2']['b']
    oec = w['oec']
    for p in w['edge_attn']:
        kk = oec @ p['k']['w'] + p['k']['b']
        vv = oec @ p['v']['w'] + p['v']['b']
        y = _attn_block(p, y, kk, vv)
    y_ref[...] = y
    u = jnp.maximum(gs[:, EMB:] + gd[:, EMB:] + y @ w['fx_wd'] + w['fx_b1'], 0.0)
    msg_ref[...] = _pad_mask(u @ w['fx']['l2']['w'] + w['fx']['l2']['b'],
                             -jnp.inf)


def _nodeupd_kernel(accs_ref, x_ref, w_ref, xo_ref, stab_ref, dtab_ref):
    w = jax.tree.map(lambda r: r[...], w_ref)
    agg2 = jnp.max(accs_ref[...], axis=0)
    agg = jnp.concatenate([agg2[:, :EMB], agg2[:, EMB:]], axis=0)
    agg = jnp.where(jnp.isneginf(agg), 0.0, agg)
    x = jnp.maximum(x_ref[...], agg)
    xo_ref[...] = x
    stab_ref[...] = jnp.concatenate([x @ w['fx_wab'], x @ w['fy_wca']], -1)
    dtab_ref[...] = jnp.concatenate([x @ w['fx_wca'], x @ w['fy_wab']], -1)


def _ym_kernel(y_ref, gs_ref, gd_ref, w_ref, yo_ref, msg_ref):
    w = jax.tree.map(lambda r: r[...], w_ref)
    gs = gs_ref[...]
    gd = gd_ref[...]
    t = jnp.maximum(gd[:, EMB:] + gs[:, EMB:] + w['fy_b1'], 0.0)
    y = jnp.maximum(y_ref[...], t @ w['fy']['l2']['w'] + w['fy']['l2']['b'])
    yo_ref[...] = y
    u = jnp.maximum(gs[:, :EMB] + gd[:, :EMB] + y @ w['fx_wd'] + w['fx_b1'], 0.0)
    msg_ref[...] = _pad_mask(u @ w['fx']['l2']['w'] + w['fx']['l2']['b'],
                             -jnp.inf)


def _yfin_kernel(y_ref, gs_ref, gd_ref, w_ref, yo_ref):
    w = jax.tree.map(lambda r: r[...], w_ref)
    t = jnp.maximum(gd_ref[...][:, EMB:] + gs_ref[...][:, EMB:] + w['fy_b1'],
                    0.0)
    yn = jnp.maximum(y_ref[...], t @ w['fy']['l2']['w'] + w['fy']['l2']['b'])
    yo_ref[...] = jnp.concatenate(
        [yn, jnp.zeros((EBLK, EMB), jnp.float32)], axis=-1)


def _wspec(tree):
    return jax.tree.map(
        lambda x: pl.BlockSpec(x.shape, lambda *_: (0,) * x.ndim), tree)


def _espec(width):
    return pl.BlockSpec((EBLK, width), lambda i: (i, 0))


def _sc_mesh():
    return plsc.VectorSubcoreMesh(core_axis_name="c", subcore_axis_name="s",
                                  num_cores=_NC, num_subcores=_NS)


@functools.lru_cache(None)
def _make_gather2():
    chunk = 512
    nchunk = _EPW // chunk

    @functools.partial(
        pl.kernel, mesh=_sc_mesh(),
        out_type=(jax.ShapeDtypeStruct((EP, 128), jnp.float32),
                  jax.ShapeDtypeStruct((EP, 128), jnp.float32)),
        scratch_types=[
            pltpu.VMEM((chunk,), jnp.int32),
            pltpu.VMEM((chunk, 128), jnp.float32),
            pltpu.SemaphoreType.DMA,
        ],
    )
    def k(stab_hbm, dtab_hbm, src_hbm, dst_hbm, gs_hbm, gd_hbm,
          idx_v, rows_v, sem):
        wid = lax.axis_index("c") * _NS + lax.axis_index("s")
        base = wid * _EPW
        for c in range(nchunk):
            off = base + c * chunk
            pltpu.sync_copy(src_hbm.at[pl.ds(off, chunk)], idx_v)
            pltpu.async_copy(stab_hbm.at[idx_v], rows_v, sem).wait()
            pltpu.sync_copy(rows_v, gs_hbm.at[pl.ds(off, chunk)])
            pltpu.sync_copy(dst_hbm.at[pl.ds(off, chunk)], idx_v)
            pltpu.async_copy(dtab_hbm.at[idx_v], rows_v, sem).wait()
            pltpu.sync_copy(rows_v, gd_hbm.at[pl.ds(off, chunk)])

    return k


@functools.lru_cache(None)
def _make_segmax():
    mchunk = 256
    nchunk = _EPW // mchunk

    @functools.partial(
        pl.kernel, mesh=_sc_mesh(),
        out_type=jax.ShapeDtypeStruct((_NW, N // 2, 2 * EMB), jnp.float32),
        scratch_types=[
            pltpu.VMEM((_EPW,), jnp.int32),
            pltpu.VMEM((mchunk, EMB), jnp.float32),
            pltpu.VMEM((N // 2, 2 * EMB), jnp.float32),
            pltpu.SemaphoreType.DMA,
        ],
    )
    def k(msg_hbm, dst_hbm, ninf_hbm, accs_hbm, idx_v, mbuf, acc, sem):
        wid = lax.axis_index("c") * _NS + lax.axis_index("s")
        base = wid * _EPW
        pltpu.sync_copy(ninf_hbm, acc)
        pltpu.sync_copy(dst_hbm.at[pl.ds(base, _EPW)], idx_v)
        for c in range(nchunk):
            pltpu.sync_copy(msg_hbm.at[pl.ds(base + c * mchunk, mchunk)], mbuf)

            def body(g, _):
                dvec = idx_v[pl.ds(c * mchunk + g * _L, _L)]
                for j in range(_L):
                    d = dvec[j]
                    hi = jnp.where(d >= N // 2, 1, 0)
                    dr = d - hi * (N // 2)
                    doff = hi * EMB
                    e = g * _L + j
                    for f in range(EMB // _L):
                        sl = pl.ds(doff + f * _L, _L)
                        msl = pl.ds(f * _L, _L)
                        acc[dr, sl] = jnp.maximum(acc[dr, sl], mbuf[e, msl])
                return 0

            lax.fori_loop(0, mchunk // _L, body, 0)
        pltpu.sync_copy(acc, accs_hbm.at[wid])

    return k


@functools.lru_cache(None)
def _make_scatter():
    @functools.partial(
        pl.kernel, mesh=_sc_mesh(),
        out_type=jax.ShapeDtypeStruct((8,), jnp.float32),
        scratch_types=[
            pltpu.VMEM((200,), jnp.int32),
            pltpu.VMEM((200, 2 * EMB), jnp.float32),
            pltpu.SemaphoreType.DMA,
        ],
    )
    def k(y_hbm, keys_hbm, base_ref, out_ref, idx_v, rows_v, sem):
        wid = lax.axis_index("c") * _NS + lax.axis_index("s")
        base = wid * _SPW
        for c in range(_SPW // 200):
            off = base + c * 200
            pltpu.sync_copy(keys_hbm.at[pl.ds(off, 200)], idx_v)
            pltpu.sync_copy(y_hbm.at[pl.ds(off, 200)], rows_v)
            pltpu.async_copy(rows_v, base_ref.at[idx_v], sem).wait()

    return k


def _gather2(stab, dtab, src_p, dst_p):
    return _make_gather2()(stab, dtab, src_p, dst_p)


def kernel(v, labels, obstacles, pos_enc, edge_index, loop, params):
    del loop
    src = edge_index[0]
    dst = edge_index[1]
    pad = jnp.zeros((EP - E,), jnp.int32)
    src_p = jnp.concatenate([src, pad])
    dst_p = jnp.concatenate([dst, pad])

    hy1 = params['hy']['l1']['w']
    fx1 = params['fx']['l1']['w']
    fy1 = params['fy']['l1']['w']
    r1 = lambda b: b.reshape(1, -1)
    rall = lambda t: jax.tree.map(lambda x: r1(x) if x.ndim == 1 else x, t)

    node_w = rall({
        'hx': params['hx'], 'onc': params['obs_node_code'],
        'oec': params['obs_edge_code'], 'node_attn': params['node_attn'],
        'hy_wab': hy1[:8] + hy1[8:16], 'hy_wca': hy1[16:24] - hy1[:8],
        'fx_wab': fx1[:EMB] + fx1[EMB:2 * EMB],
        'fx_wca': fx1[2 * EMB:3 * EMB] - fx1[:EMB],
    })
    nupd_w = {
        'fx_wab': fx1[:EMB] + fx1[EMB:2 * EMB],
        'fx_wca': fx1[2 * EMB:3 * EMB] - fx1[:EMB],
        'fy_wab': fy1[:EMB] + fy1[EMB:2 * EMB],
        'fy_wca': fy1[2 * EMB:3 * EMB] - fy1[:EMB],
    }
    ym_w = rall({
        'fy': params['fy'], 'fy_b1': r1(params['fy']['l1']['b']),
        'fx': params['fx'], 'fx_b1': r1(params['fx']['l1']['b']),
        'fx_wd': fx1[3 * EMB:],
    })
    yfin_w = {'fy': ym_w['fy'], 'fy_b1': ym_w['fy_b1']}

    f32 = jnp.float32
    x0, stab, dtab, oec, keys = pl.pallas_call(
        _node_kernel,
        out_shape=(jax.ShapeDtypeStruct((N, EMB), f32),
                   jax.ShapeDtypeStruct((N, 2 * EMB), f32),
                   jax.ShapeDtypeStruct((N, 2 * EMB), f32),
                   jax.ShapeDtypeStruct((NOBS, EMB), f32),
                   jax.ShapeDtypeStruct((8, E // 8), jnp.int32)),
    )(v, labels, obstacles, pos_enc,
      src.reshape(8, E // 8), dst.reshape(8, E // 8), node_w)
    keys = keys.reshape(E)
    base = jax.new_ref(jnp.zeros((N * N, 2 * EMB), f32))

    gs, gd = _gather2(stab, dtab, src_p, dst_p)

    edge_w = rall({
        'hy': params['hy'], 'hy_b1': r1(params['hy']['l1']['b']),
        'edge_attn': params['edge_attn'], 'oec': oec,
        'fx': params['fx'], 'fx_b1': r1(params['fx']['l1']['b']),
        'fx_wd': fx1[3 * EMB:],
    })
    y, msg = pl.pallas_call(
        _edge_kernel,
        grid=(NGRID,),
        in_specs=[_espec(2 * EMB), _espec(2 * EMB), _wspec(edge_w)],
        out_specs=(_espec(EMB), _espec(EMB)),
        out_shape=(jax.ShapeDtypeStruct((EP, EMB), f32),
                   jax.ShapeDtypeStruct((EP, EMB), f32)),
    )(gs, gd, edge_w)

    ninf = jnp.full((N // 2, 2 * EMB), -jnp.inf, f32)
    nupd_call = pl.pallas_call(
        _nodeupd_kernel,
        out_shape=(jax.ShapeDtypeStruct((N, EMB), f32),
                   jax.ShapeDtypeStruct((N, 2 * EMB), f32),
                   jax.ShapeDtypeStruct((N, 2 * EMB), f32)),
    )
    ym_call = pl.pallas_call(
        _ym_kernel,
        grid=(NGRID,),
        in_specs=[_espec(EMB), _espec(2 * EMB), _espec(2 * EMB), _wspec(ym_w)],
        out_specs=(_espec(EMB), _espec(EMB)),
        out_shape=(jax.ShapeDtypeStruct((EP, EMB), f32),
                   jax.ShapeDtypeStruct((EP, EMB), f32)),
    )

    x = x0
    for it in range(3):
        accs = _make_segmax()(msg, dst_p, ninf)
        x, stab, dtab = nupd_call(accs, x, nupd_w)
        gs, gd = _gather2(stab, dtab, src_p, dst_p)
        if it < 2:
            y, msg = ym_call(y, gs, gd, ym_w)
        else:
            y = pl.pallas_call(
                _yfin_kernel,
                grid=(NGRID,),
                in_specs=[_espec(EMB), _espec(2 * EMB), _espec(2 * EMB),
                          _wspec(yfin_w)],
                out_specs=_espec(2 * EMB),
                out_shape=jax.ShapeDtypeStruct((EP, 2 * EMB), f32),
            )(y, gs, gd, yfin_w)

    _make_scatter()(y, keys, base)
    flat = base[...][:, :EMB]
    return (flat.reshape(N, N, EMB), x)


# speedup vs baseline: 1.1376x; 1.1374x over previous
import functools

import jax
import jax.numpy as jnp
import numpy as np
from jax import lax
from jax.experimental import pallas as pl
from jax.experimental.pallas import tpu as pltpu
from jax.experimental.pallas import tpu_sc as plsc

N = 1000
E = 32000
EP = 32768
EMB = 64
OBS = 6
NOBS = 32

EBLK = 4096
NGRID = EP // EBLK

_NC, _NS, _L = 2, 16, 16
_NW = _NC * _NS
_EPW = EP // _NW
_SPW = E // _NW


def _ln(g, b, x):
    m = x.mean(-1, keepdims=True)
    v = ((x - m) ** 2).mean(-1, keepdims=True)
    return (x - m) / jnp.sqrt(v + 1e-5) * g + b


def _attn_block(p, x, kk, vv):
    q = x @ p['q']['w'] + p['q']['b']
    s = q @ kk.T / np.sqrt(EMB)
    s = s - s.max(-1, keepdims=True)
    es = jnp.exp(s)
    attn = es / es.sum(-1, keepdims=True)
    x = _ln(p['ln1']['g'], p['ln1']['b'], x + attn @ vv)
    h = jnp.maximum(x @ p['mlp']['l1']['w'] + p['mlp']['l1']['b'], 0.0)
    h = h @ p['mlp']['l2']['w'] + p['mlp']['l2']['b']
    return _ln(p['ln2']['g'], p['ln2']['b'], x + h)


def _pad_mask(v, fill):
    i = pl.program_id(0)
    rows = lax.broadcasted_iota(jnp.int32, v.shape, 0) + i * EBLK
    return jnp.where(rows < E, v, fill)


def _node_kernel(v_ref, lab_ref, obs_ref, pe_ref, src_ref, dst_ref, w_ref,
                 x_ref, stab_ref, dtab_ref, oec_ref, keys_ref):
    w = jax.tree.map(lambda r: r[...], w_ref)
    vc = jnp.concatenate([v_ref[...], lab_ref[...]], axis=-1)
    lab0 = lab_ref[...][:, 0]
    gw = (lab0 == 1.0).astype(jnp.float32)
    goal = gw[None, :] @ vc
    d = vc - goal
    feat = jnp.concatenate([vc, jnp.broadcast_to(goal, vc.shape), d, d * d], -1)
    h = jnp.maximum(feat @ w['hx']['l1']['w'] + w['hx']['l1']['b'], 0.0)
    x = h @ w['hx']['l2']['w'] + w['hx']['l2']['b']

    obs = obs_ref[...]
    pe = pe_ref[...]
    h = jnp.maximum(obs @ w['onc']['l1']['w'] + w['onc']['l1']['b'], 0.0)
    onc = h @ w['onc']['l2']['w'] + w['onc']['l2']['b'] + pe
    h = jnp.maximum(obs @ w['oec']['l1']['w'] + w['oec']['l1']['b'], 0.0)
    oec = h @ w['oec']['l2']['w'] + w['oec']['l2']['b'] + pe
    oec_ref[...] = oec

    for p in w['node_attn']:
        kk = onc @ p['k']['w'] + p['k']['b']
        vv = onc @ p['v']['w'] + p['v']['b']
        x = _attn_block(p, x, kk, vv)
    x_ref[...] = x

    q_t = vc @ w['hy_wca']
    p_t = vc @ w['hy_wab']
    a0 = x @ w['fx_wab']
    b0 = x @ w['fx_wca']
    stab_ref[...] = jnp.concatenate([q_t, a0], axis=-1)
    dtab_ref[...] = jnp.concatenate([p_t, b0], axis=-1)
    keys_ref[...] = src_ref[...] * N + dst_ref[...]


def _edge_kernel(gs_ref, gd_ref, w_ref, y_ref, msg_ref):
    w = jax.tree.map(lambda r: r[...], w_ref)
    gs = gs_ref[...]
    gd = gd_ref[...]
    h = jnp.maximum(gd[:, :EMB] + gs[:, :EMB] + w['hy_b1'], 0.0)
    y = h @ w['hy']['l2']['w'] + w['hy']['l2']['b']
    oec = w['oec']
    for p in w['edge_attn']:
        kk = oec @ p['k']['w'] + p['k']['b']
        vv = oec @ p['v']['w'] + p['v']['b']
        y = _attn_block(p, y, kk, vv)
    y_ref[...] = y
    u = jnp.maximum(gs[:, EMB:] + gd[:, EMB:] + y @ w['fx_wd'] + w['fx_b1'], 0.0)
    msg_ref[...] = _pad_mask(u @ w['fx']['l2']['w'] + w['fx']['l2']['b'],
                             -jnp.inf)


def _nodeupd_kernel(accs_ref, x_ref, w_ref, xo_ref, stab_ref, dtab_ref):
    w = jax.tree.map(lambda r: r[...], w_ref)
    agg2 = jnp.max(accs_ref[...], axis=0)
    agg = jnp.concatenate([agg2[:, :EMB], agg2[:, EMB:]], axis=0)
    agg = jnp.where(jnp.isneginf(agg), 0.0, agg)
    x = jnp.maximum(x_ref[...], agg)
    xo_ref[...] = x
    stab_ref[...] = jnp.concatenate([x @ w['fx_wab'], x @ w['fy_wca']], -1)
    dtab_ref[...] = jnp.concatenate([x @ w['fx_wca'], x @ w['fy_wab']], -1)


def _ym_kernel(y_ref, gs_ref, gd_ref, w_ref, yo_ref, msg_ref):
    w = jax.tree.map(lambda r: r[...], w_ref)
    gs = gs_ref[...]
    gd = gd_ref[...]
    t = jnp.maximum(gd[:, EMB:] + gs[:, EMB:] + w['fy_b1'], 0.0)
    y = jnp.maximum(y_ref[...], t @ w['fy']['l2']['w'] + w['fy']['l2']['b'])
    yo_ref[...] = y
    u = jnp.maximum(gs[:, :EMB] + gd[:, :EMB] + y @ w['fx_wd'] + w['fx_b1'], 0.0)
    msg_ref[...] = _pad_mask(u @ w['fx']['l2']['w'] + w['fx']['l2']['b'],
                             -jnp.inf)


def _yfin_kernel(y_ref, gs_ref, gd_ref, w_ref, yo_ref):
    w = jax.tree.map(lambda r: r[...], w_ref)
    t = jnp.maximum(gd_ref[...][:, EMB:] + gs_ref[...][:, EMB:] + w['fy_b1'],
                    0.0)
    yn = jnp.maximum(y_ref[...], t @ w['fy']['l2']['w'] + w['fy']['l2']['b'])
    yo_ref[...] = jnp.concatenate(
        [yn, jnp.zeros((EBLK, EMB), jnp.float32)], axis=-1)


def _wspec(tree):
    return jax.tree.map(
        lambda x: pl.BlockSpec(x.shape, lambda *_: (0,) * x.ndim), tree)


def _espec(width):
    return pl.BlockSpec((EBLK, width), lambda i: (i, 0))


def _sc_mesh():
    return plsc.VectorSubcoreMesh(core_axis_name="c", subcore_axis_name="s",
                                  num_cores=_NC, num_subcores=_NS)


@functools.lru_cache(None)
def _make_gather2():
    chunk = 256
    nchunk = _EPW // chunk

    @functools.partial(
        pl.kernel, mesh=_sc_mesh(),
        out_type=(jax.ShapeDtypeStruct((EP, 128), jnp.float32),
                  jax.ShapeDtypeStruct((EP, 128), jnp.float32)),
        scratch_types=[
            pltpu.VMEM((chunk,), jnp.int32),
            pltpu.VMEM((chunk,), jnp.int32),
            pltpu.VMEM((chunk, 128), jnp.float32),
            pltpu.VMEM((chunk, 128), jnp.float32),
            pltpu.SemaphoreType.DMA,
            pltpu.SemaphoreType.DMA,
        ],
    )
    def k(stab_hbm, dtab_hbm, src_hbm, dst_hbm, gs_hbm, gd_hbm,
          idxs_v, idxd_v, rows_v, rowd_v, sems, semd):
        wid = lax.axis_index("c") * _NS + lax.axis_index("s")
        base = wid * _EPW
        for c in range(nchunk):
            off = base + c * chunk
            pltpu.sync_copy(src_hbm.at[pl.ds(off, chunk)], idxs_v)
            pltpu.sync_copy(dst_hbm.at[pl.ds(off, chunk)], idxd_v)
            g1 = pltpu.async_copy(stab_hbm.at[idxs_v], rows_v, sems)
            g2 = pltpu.async_copy(dtab_hbm.at[idxd_v], rowd_v, semd)
            g1.wait()
            pltpu.sync_copy(rows_v, gs_hbm.at[pl.ds(off, chunk)])
            g2.wait()
            pltpu.sync_copy(rowd_v, gd_hbm.at[pl.ds(off, chunk)])

    return k


@functools.lru_cache(None)
def _make_segmax():
    mchunk = 256
    nchunk = _EPW // mchunk

    @functools.partial(
        pl.kernel, mesh=_sc_mesh(),
        out_type=jax.ShapeDtypeStruct((_NW, N // 2, 2 * EMB), jnp.float32),
        scratch_types=[
            pltpu.VMEM((_EPW,), jnp.int32),
            pltpu.VMEM((mchunk, EMB), jnp.float32),
            pltpu.VMEM((N // 2, 2 * EMB), jnp.float32),
            pltpu.SemaphoreType.DMA,
        ],
    )
    def k(msg_hbm, dst_hbm, ninf_hbm, accs_hbm, idx_v, mbuf, acc, sem):
        wid = lax.axis_index("c") * _NS + lax.axis_index("s")
        base = wid * _EPW
        pltpu.sync_copy(ninf_hbm, acc)
        pltpu.sync_copy(dst_hbm.at[pl.ds(base, _EPW)], idx_v)
        for c in range(nchunk):
            pltpu.sync_copy(msg_hbm.at[pl.ds(base + c * mchunk, mchunk)], mbuf)

            def body(g, _):
                dvec = idx_v[pl.ds(c * mchunk + g * _L, _L)]
                for j in range(_L):
                    d = dvec[j]
                    hi = jnp.where(d >= N // 2, 1, 0)
                    dr = d - hi * (N // 2)
                    doff = hi * EMB
                    e = g * _L + j
                    for f in range(EMB // _L):
                        sl = pl.ds(doff + f * _L, _L)
                        msl = pl.ds(f * _L, _L)
                        acc[dr, sl] = jnp.maximum(acc[dr, sl], mbuf[e, msl])
                return 0

            lax.fori_loop(0, mchunk // _L, body, 0)
        pltpu.sync_copy(acc, accs_hbm.at[wid])

    return k


@functools.lru_cache(None)
def _make_scatter():
    @functools.partial(
        pl.kernel, mesh=_sc_mesh(),
        out_type=jax.ShapeDtypeStruct((8,), jnp.float32),
        scratch_types=[
            pltpu.VMEM((200,), jnp.int32),
            pltpu.VMEM((200, 2 * EMB), jnp.float32),
            pltpu.SemaphoreType.DMA,
        ],
    )
    def k(y_hbm, keys_hbm, base_ref, out_ref, idx_v, rows_v, sem):
        wid = lax.axis_index("c") * _NS + lax.axis_index("s")
        base = wid * _SPW
        for c in range(_SPW // 200):
            off = base + c * 200
            pltpu.sync_copy(keys_hbm.at[pl.ds(off, 200)], idx_v)
            pltpu.sync_copy(y_hbm.at[pl.ds(off, 200)], rows_v)
            pltpu.async_copy(rows_v, base_ref.at[idx_v], sem).wait()

    return k


def _gather2(stab, dtab, src_p, dst_p):
    return _make_gather2()(stab, dtab, src_p, dst_p)


def kernel(v, labels, obstacles, pos_enc, edge_index, loop, params):
    del loop
    src = edge_index[0]
    dst = edge_index[1]
    pad = jnp.zeros((EP - E,), jnp.int32)
    src_p = jnp.concatenate([src, pad])
    dst_p = jnp.concatenate([dst, pad])

    hy1 = params['hy']['l1']['w']
    fx1 = params['fx']['l1']['w']
    fy1 = params['fy']['l1']['w']
    r1 = lambda b: b.reshape(1, -1)
    rall = lambda t: jax.tree.map(lambda x: r1(x) if x.ndim == 1 else x, t)

    node_w = rall({
        'hx': params['hx'], 'onc': params['obs_node_code'],
        'oec': params['obs_edge_code'], 'node_attn': params['node_attn'],
        'hy_wab': hy1[:8] + hy1[8:16], 'hy_wca': hy1[16:24] - hy1[:8],
        'fx_wab': fx1[:EMB] + fx1[EMB:2 * EMB],
        'fx_wca': fx1[2 * EMB:3 * EMB] - fx1[:EMB],
    })
    nupd_w = {
        'fx_wab': fx1[:EMB] + fx1[EMB:2 * EMB],
        'fx_wca': fx1[2 * EMB:3 * EMB] - fx1[:EMB],
        'fy_wab': fy1[:EMB] + fy1[EMB:2 * EMB],
        'fy_wca': fy1[2 * EMB:3 * EMB] - fy1[:EMB],
    }
    ym_w = rall({
        'fy': params['fy'], 'fy_b1': r1(params['fy']['l1']['b']),
        'fx': params['fx'], 'fx_b1': r1(params['fx']['l1']['b']),
        'fx_wd': fx1[3 * EMB:],
    })
    yfin_w = {'fy': ym_w['fy'], 'fy_b1': ym_w['fy_b1']}

    f32 = jnp.float32
    x0, stab, dtab, oec, keys = pl.pallas_call(
        _node_kernel,
        out_shape=(jax.ShapeDtypeStruct((N, EMB), f32),
                   jax.ShapeDtypeStruct((N, 2 * EMB), f32),
                   jax.ShapeDtypeStruct((N, 2 * EMB), f32),
                   jax.ShapeDtypeStruct((NOBS, EMB), f32),
                   jax.ShapeDtypeStruct((8, E // 8), jnp.int32)),
    )(v, labels, obstacles, pos_enc,
      src.reshape(8, E // 8), dst.reshape(8, E // 8), node_w)
    keys = keys.reshape(E)
    base = jax.new_ref(jnp.zeros((N * N, 2 * EMB), f32))

    gs, gd = _gather2(stab, dtab, src_p, dst_p)

    edge_w = rall({
        'hy': params['hy'], 'hy_b1': r1(params['hy']['l1']['b']),
        'edge_attn': params['edge_attn'], 'oec': oec,
        'fx': params['fx'], 'fx_b1': r1(params['fx']['l1']['b']),
        'fx_wd': fx1[3 * EMB:],
    })
    y, msg = pl.pallas_call(
        _edge_kernel,
        grid=(NGRID,),
        in_specs=[_espec(2 * EMB), _espec(2 * EMB), _wspec(edge_w)],
        out_specs=(_espec(EMB), _espec(EMB)),
        out_shape=(jax.ShapeDtypeStruct((EP, EMB), f32),
                   jax.ShapeDtypeStruct((EP, EMB), f32)),
    )(gs, gd, edge_w)

    ninf = jnp.full((N // 2, 2 * EMB), -jnp.inf, f32)
    nupd_call = pl.pallas_call(
        _nodeupd_kernel,
        out_shape=(jax.ShapeDtypeStruct((N, EMB), f32),
                   jax.ShapeDtypeStruct((N, 2 * EMB), f32),
                   jax.ShapeDtypeStruct((N, 2 * EMB), f32)),
    )
    ym_call = pl.pallas_call(
        _ym_kernel,
        grid=(NGRID,),
        in_specs=[_espec(EMB), _espec(2 * EMB), _espec(2 * EMB), _wspec(ym_w)],
        out_specs=(_espec(EMB), _espec(EMB)),
        out_shape=(jax.ShapeDtypeStruct((EP, EMB), f32),
                   jax.ShapeDtypeStruct((EP, EMB), f32)),
    )

    x = x0
    for it in range(3):
        accs = _make_segmax()(msg, dst_p, ninf)
        x, stab, dtab = nupd_call(accs, x, nupd_w)
        gs, gd = _gather2(stab, dtab, src_p, dst_p)
        if it < 2:
            y, msg = ym_call(y, gs, gd, ym_w)
        else:
            y = pl.pallas_call(
                _yfin_kernel,
                grid=(NGRID,),
                in_specs=[_espec(EMB), _espec(2 * EMB), _espec(2 * EMB),
                          _wspec(yfin_w)],
                out_specs=_espec(2 * EMB),
                out_shape=jax.ShapeDtypeStruct((EP, 2 * EMB), f32),
            )(y, gs, gd, yfin_w)

    _make_scatter()(y, keys, base)
    flat = base[...][:, :EMB]
    return (flat.reshape(N, N, EMB), x)


# speedup vs baseline: 1.1732x; 1.0313x over previous
import functools

import jax
import jax.numpy as jnp
import numpy as np
from jax import lax
from jax.experimental import pallas as pl
from jax.experimental.pallas import tpu as pltpu
from jax.experimental.pallas import tpu_sc as plsc

N = 1000
E = 32000
EP = 32768
EMB = 64
OBS = 6
NOBS = 32

EBLK = 4096
NGRID = EP // EBLK

_NC, _NS, _L = 2, 16, 16
_NW = _NC * _NS
_EPW = EP // _NW
_SPW = E // _NW


def _ln(g, b, x):
    m = x.mean(-1, keepdims=True)
    v = ((x - m) ** 2).mean(-1, keepdims=True)
    return (x - m) / jnp.sqrt(v + 1e-5) * g + b


def _attn_block(p, x, kk, vv):
    q = x @ p['q']['w'] + p['q']['b']
    s = q @ kk.T / np.sqrt(EMB)
    s = s - s.max(-1, keepdims=True)
    es = jnp.exp(s)
    attn = es / es.sum(-1, keepdims=True)
    x = _ln(p['ln1']['g'], p['ln1']['b'], x + attn @ vv)
    h = jnp.maximum(x @ p['mlp']['l1']['w'] + p['mlp']['l1']['b'], 0.0)
    h = h @ p['mlp']['l2']['w'] + p['mlp']['l2']['b']
    return _ln(p['ln2']['g'], p['ln2']['b'], x + h)


def _pad_mask(v, fill):
    i = pl.program_id(0)
    rows = lax.broadcasted_iota(jnp.int32, v.shape, 0) + i * EBLK
    return jnp.where(rows < E, v, fill)


def _node_kernel(v_ref, lab_ref, obs_ref, pe_ref, src_ref, dst_ref, w_ref,
                 x_ref, stab_ref, dtab_ref, oec_ref, keys_ref):
    w = jax.tree.map(lambda r: r[...], w_ref)
    vc = jnp.concatenate([v_ref[...], lab_ref[...]], axis=-1)
    lab0 = lab_ref[...][:, 0]
    gw = (lab0 == 1.0).astype(jnp.float32)
    goal = gw[None, :] @ vc
    d = vc - goal
    feat = jnp.concatenate([vc, jnp.broadcast_to(goal, vc.shape), d, d * d], -1)
    h = jnp.maximum(feat @ w['hx']['l1']['w'] + w['hx']['l1']['b'], 0.0)
    x = h @ w['hx']['l2']['w'] + w['hx']['l2']['b']

    obs = obs_ref[...]
    pe = pe_ref[...]
    h = jnp.maximum(obs @ w['onc']['l1']['w'] + w['onc']['l1']['b'], 0.0)
    onc = h @ w['onc']['l2']['w'] + w['onc']['l2']['b'] + pe
    h = jnp.maximum(obs @ w['oec']['l1']['w'] + w['oec']['l1']['b'], 0.0)
    oec = h @ w['oec']['l2']['w'] + w['oec']['l2']['b'] + pe
    oec_ref[...] = oec

    for p in w['node_attn']:
        kk = onc @ p['k']['w'] + p['k']['b']
        vv = onc @ p['v']['w'] + p['v']['b']
        x = _attn_block(p, x, kk, vv)
    x_ref[...] = x

    q_t = vc @ w['hy_wca']
    p_t = vc @ w['hy_wab']
    a0 = x @ w['fx_wab']
    b0 = x @ w['fx_wca']
    stab_ref[...] = jnp.concatenate([q_t, a0], axis=-1)
    dtab_ref[...] = jnp.concatenate([p_t, b0], axis=-1)
    keys_ref[...] = src_ref[...] * N + dst_ref[...]


def _edge_kernel(gs_ref, gd_ref, w_ref, y_ref, msg_ref):
    w = jax.tree.map(lambda r: r[...], w_ref)
    gs = gs_ref[...]
    gd = gd_ref[...]
    h = jnp.maximum(gd[:, :EMB] + gs[:, :EMB] + w['hy_b1'], 0.0)
    y = h @ w['hy']['l2']['w'] + w['hy']['l2']['b']
    oec = w['oec']
    for p in w['edge_attn']:
        kk = oec @ p['k']['w'] + p['k']['b']
        vv = oec @ p['v']['w'] + p['v']['b']
        y = _attn_block(p, y, kk, vv)
    y_ref[...] = y
    u = jnp.maximum(gs[:, EMB:] + gd[:, EMB:] + y @ w['fx_wd'] + w['fx_b1'], 0.0)
    msg_ref[...] = _pad_mask(u @ w['fx']['l2']['w'] + w['fx']['l2']['b'],
                             -jnp.inf)


def _nodeupd_kernel(accs_ref, x_ref, w_ref, xo_ref, stab_ref, dtab_ref):
    w = jax.tree.map(lambda r: r[...], w_ref)
    agg2 = jnp.max(accs_ref[...], axis=0)
    agg = jnp.concatenate([agg2[:, :EMB], agg2[:, EMB:]], axis=0)
    agg = jnp.where(jnp.isneginf(agg), 0.0, agg)
    x = jnp.maximum(x_ref[...], agg)
    xo_ref[...] = x
    stab_ref[...] = jnp.concatenate([x @ w['fx_wab'], x @ w['fy_wca']], -1)
    dtab_ref[...] = jnp.concatenate([x @ w['fx_wca'], x @ w['fy_wab']], -1)


def _ym_kernel(y_ref, gs_ref, gd_ref, w_ref, yo_ref, msg_ref):
    w = jax.tree.map(lambda r: r[...], w_ref)
    gs = gs_ref[...]
    gd = gd_ref[...]
    t = jnp.maximum(gd[:, EMB:] + gs[:, EMB:] + w['fy_b1'], 0.0)
    y = jnp.maximum(y_ref[...], t @ w['fy']['l2']['w'] + w['fy']['l2']['b'])
    yo_ref[...] = y
    u = jnp.maximum(gs[:, :EMB] + gd[:, :EMB] + y @ w['fx_wd'] + w['fx_b1'], 0.0)
    msg_ref[...] = _pad_mask(u @ w['fx']['l2']['w'] + w['fx']['l2']['b'],
                             -jnp.inf)


def _yfin_kernel(y_ref, gs_ref, gd_ref, w_ref, yo_ref):
    w = jax.tree.map(lambda r: r[...], w_ref)
    t = jnp.maximum(gd_ref[...][:, EMB:] + gs_ref[...][:, EMB:] + w['fy_b1'],
                    0.0)
    yn = jnp.maximum(y_ref[...], t @ w['fy']['l2']['w'] + w['fy']['l2']['b'])
    yo_ref[...] = jnp.concatenate(
        [yn, jnp.zeros((EBLK, EMB), jnp.float32)], axis=-1)


def _wspec(tree):
    return jax.tree.map(
        lambda x: pl.BlockSpec(x.shape, lambda *_: (0,) * x.ndim), tree)


def _espec(width):
    return pl.BlockSpec((EBLK, width), lambda i: (i, 0))


def _sc_mesh():
    return plsc.VectorSubcoreMesh(core_axis_name="c", subcore_axis_name="s",
                                  num_cores=_NC, num_subcores=_NS)


@functools.lru_cache(None)
def _make_gather2():
    chunk = 128
    nchunk = _EPW // chunk

    @functools.partial(
        pl.kernel, mesh=_sc_mesh(),
        out_type=(jax.ShapeDtypeStruct((EP, 128), jnp.float32),
                  jax.ShapeDtypeStruct((EP, 128), jnp.float32)),
        scratch_types=[
            pltpu.VMEM((_EPW,), jnp.int32),
            pltpu.VMEM((_EPW,), jnp.int32),
            [pltpu.VMEM((chunk, 128), jnp.float32)] * 2,
            [pltpu.VMEM((chunk, 128), jnp.float32)] * 2,
            [pltpu.SemaphoreType.DMA] * 2,
            [pltpu.SemaphoreType.DMA] * 2,
        ],
    )
    def k(stab_hbm, dtab_hbm, src_hbm, dst_hbm, gs_hbm, gd_hbm,
          idxs_v, idxd_v, rows_v, rowd_v, sems, semd):
        wid = lax.axis_index("c") * _NS + lax.axis_index("s")
        base = wid * _EPW
        pltpu.sync_copy(src_hbm.at[pl.ds(base, _EPW)], idxs_v)
        pltpu.sync_copy(dst_hbm.at[pl.ds(base, _EPW)], idxd_v)

        def start(c, b):
            sl = pl.ds(c * chunk, chunk)
            g1 = pltpu.async_copy(stab_hbm.at[idxs_v.at[sl]], rows_v[b],
                                  sems[b])
            g2 = pltpu.async_copy(dtab_hbm.at[idxd_v.at[sl]], rowd_v[b],
                                  semd[b])
            return g1, g2

        pend = start(0, 0)
        for c in range(nchunk):
            b = c % 2
            g1, g2 = pend
            if c + 1 < nchunk:
                nxt = start(c + 1, (c + 1) % 2)
            off = base + c * chunk
            g1.wait()
            pltpu.sync_copy(rows_v[b], gs_hbm.at[pl.ds(off, chunk)])
            g2.wait()
            pltpu.sync_copy(rowd_v[b], gd_hbm.at[pl.ds(off, chunk)])
            if c + 1 < nchunk:
                pend = nxt

    return k


@functools.lru_cache(None)
def _make_segmax():
    mchunk = 256
    nchunk = _EPW // mchunk

    @functools.partial(
        pl.kernel, mesh=_sc_mesh(),
        out_type=jax.ShapeDtypeStruct((_NW, N // 2, 2 * EMB), jnp.float32),
        scratch_types=[
            pltpu.VMEM((_EPW,), jnp.int32),
            pltpu.VMEM((mchunk, EMB), jnp.float32),
            pltpu.VMEM((N // 2, 2 * EMB), jnp.float32),
            pltpu.SemaphoreType.DMA,
        ],
    )
    def k(msg_hbm, dst_hbm, ninf_hbm, accs_hbm, idx_v, mbuf, acc, sem):
        wid = lax.axis_index("c") * _NS + lax.axis_index("s")
        base = wid * _EPW
        pltpu.sync_copy(ninf_hbm, acc)
        pltpu.sync_copy(dst_hbm.at[pl.ds(base, _EPW)], idx_v)
        for c in range(nchunk):
            pltpu.sync_copy(msg_hbm.at[pl.ds(base + c * mchunk, mchunk)], mbuf)

            def body(g, _):
                dvec = idx_v[pl.ds(c * mchunk + g * _L, _L)]
                for j in range(_L):
                    d = dvec[j]
                    hi = jnp.where(d >= N // 2, 1, 0)
                    dr = d - hi * (N // 2)
                    doff = hi * EMB
                    e = g * _L + j
                    for f in range(EMB // _L):
                        sl = pl.ds(doff + f * _L, _L)
                        msl = pl.ds(f * _L, _L)
                        acc[dr, sl] = jnp.maximum(acc[dr, sl], mbuf[e, msl])
                return 0

            lax.fori_loop(0, mchunk // _L, body, 0)
        pltpu.sync_copy(acc, accs_hbm.at[wid])

    return k


@functools.lru_cache(None)
def _make_scatter():
    @functools.partial(
        pl.kernel, mesh=_sc_mesh(),
        out_type=jax.ShapeDtypeStruct((8,), jnp.float32),
        scratch_types=[
            pltpu.VMEM((200,), jnp.int32),
            pltpu.VMEM((200, 2 * EMB), jnp.float32),
            pltpu.SemaphoreType.DMA,
        ],
    )
    def k(y_hbm, keys_hbm, base_ref, out_ref, idx_v, rows_v, sem):
        wid = lax.axis_index("c") * _NS + lax.axis_index("s")
        base = wid * _SPW
        for c in range(_SPW // 200):
            off = base + c * 200
            pltpu.sync_copy(keys_hbm.at[pl.ds(off, 200)], idx_v)
            pltpu.sync_copy(y_hbm.at[pl.ds(off, 200)], rows_v)
            pltpu.async_copy(rows_v, base_ref.at[idx_v], sem).wait()

    return k


def _gather2(stab, dtab, src_p, dst_p):
    return _make_gather2()(stab, dtab, src_p, dst_p)


def kernel(v, labels, obstacles, pos_enc, edge_index, loop, params):
    del loop
    src = edge_index[0]
    dst = edge_index[1]
    pad = jnp.zeros((EP - E,), jnp.int32)
    src_p = jnp.concatenate([src, pad])
    dst_p = jnp.concatenate([dst, pad])

    hy1 = params['hy']['l1']['w']
    fx1 = params['fx']['l1']['w']
    fy1 = params['fy']['l1']['w']
    r1 = lambda b: b.reshape(1, -1)
    rall = lambda t: jax.tree.map(lambda x: r1(x) if x.ndim == 1 else x, t)

    node_w = rall({
        'hx': params['hx'], 'onc': params['obs_node_code'],
        'oec': params['obs_edge_code'], 'node_attn': params['node_attn'],
        'hy_wab': hy1[:8] + hy1[8:16], 'hy_wca': hy1[16:24] - hy1[:8],
        'fx_wab': fx1[:EMB] + fx1[EMB:2 * EMB],
        'fx_wca': fx1[2 * EMB:3 * EMB] - fx1[:EMB],
    })
    nupd_w = {
        'fx_wab': fx1[:EMB] + fx1[EMB:2 * EMB],
        'fx_wca': fx1[2 * EMB:3 * EMB] - fx1[:EMB],
        'fy_wab': fy1[:EMB] + fy1[EMB:2 * EMB],
        'fy_wca': fy1[2 * EMB:3 * EMB] - fy1[:EMB],
    }
    ym_w = rall({
        'fy': params['fy'], 'fy_b1': r1(params['fy']['l1']['b']),
        'fx': params['fx'], 'fx_b1': r1(params['fx']['l1']['b']),
        'fx_wd': fx1[3 * EMB:],
    })
    yfin_w = {'fy': ym_w['fy'], 'fy_b1': ym_w['fy_b1']}

    f32 = jnp.float32
    x0, stab, dtab, oec, keys = pl.pallas_call(
        _node_kernel,
        out_shape=(jax.ShapeDtypeStruct((N, EMB), f32),
                   jax.ShapeDtypeStruct((N, 2 * EMB), f32),
                   jax.ShapeDtypeStruct((N, 2 * EMB), f32),
                   jax.ShapeDtypeStruct((NOBS, EMB), f32),
                   jax.ShapeDtypeStruct((8, E // 8), jnp.int32)),
    )(v, labels, obstacles, pos_enc,
      src.reshape(8, E // 8), dst.reshape(8, E // 8), node_w)
    keys = keys.reshape(E)
    base = jax.new_ref(jnp.zeros((N * N, 2 * EMB), f32))

    gs, gd = _gather2(stab, dtab, src_p, dst_p)

    edge_w = rall({
        'hy': params['hy'], 'hy_b1': r1(params['hy']['l1']['b']),
        'edge_attn': params['edge_attn'], 'oec': oec,
        'fx': params['fx'], 'fx_b1': r1(params['fx']['l1']['b']),
        'fx_wd': fx1[3 * EMB:],
    })
    y, msg = pl.pallas_call(
        _edge_kernel,
        grid=(NGRID,),
        in_specs=[_espec(2 * EMB), _espec(2 * EMB), _wspec(edge_w)],
        out_specs=(_espec(EMB), _espec(EMB)),
        out_shape=(jax.ShapeDtypeStruct((EP, EMB), f32),
                   jax.ShapeDtypeStruct((EP, EMB), f32)),
    )(gs, gd, edge_w)

    ninf = jnp.full((N // 2, 2 * EMB), -jnp.inf, f32)
    nupd_call = pl.pallas_call(
        _nodeupd_kernel,
        out_shape=(jax.ShapeDtypeStruct((N, EMB), f32),
                   jax.ShapeDtypeStruct((N, 2 * EMB), f32),
                   jax.ShapeDtypeStruct((N, 2 * EMB), f32)),
    )
    ym_call = pl.pallas_call(
        _ym_kernel,
        grid=(NGRID,),
        in_specs=[_espec(EMB), _espec(2 * EMB), _espec(2 * EMB), _wspec(ym_w)],
        out_specs=(_espec(EMB), _espec(EMB)),
        out_shape=(jax.ShapeDtypeStruct((EP, EMB), f32),
                   jax.ShapeDtypeStruct((EP, EMB), f32)),
    )

    x = x0
    for it in range(3):
        accs = _make_segmax()(msg, dst_p, ninf)
        x, stab, dtab = nupd_call(accs, x, nupd_w)
        gs, gd = _gather2(stab, dtab, src_p, dst_p)
        if it < 2:
            y, msg = ym_call(y, gs, gd, ym_w)
        else:
            y = pl.pallas_call(
                _yfin_kernel,
                grid=(NGRID,),
                in_specs=[_espec(EMB), _espec(2 * EMB), _espec(2 * EMB),
                          _wspec(yfin_w)],
                out_specs=_espec(2 * EMB),
                out_shape=jax.ShapeDtypeStruct((EP, 2 * EMB), f32),
            )(y, gs, gd, yfin_w)

    _make_scatter()(y, keys, base)
    flat = base[...][:, :EMB]
    return (flat.reshape(N, N, EMB), x)
